# Initial kernel scaffold; baseline (speedup 1.0000x reference)
import jax
import jax.numpy as jnp
from jax.experimental import pallas as pl


def kernel(etype, rid, att_rc, att_rp, W_type, W_rid, W_rc, b_rc, W_rp, b_rp):
    raise NotImplementedError("write your pallas kernel here")


# speedup vs baseline: 1.5299x; 1.5299x over previous
import jax
import jax.numpy as jnp
from jax import lax
from jax.experimental import pallas as pl
from jax.experimental.pallas import tpu as pltpu
from jax.experimental.pallas import tpu_sc as plsc

E = 320000
HID = 128
CHUNK = 128
NCHUNKS = E // CHUNK
NC, NS = 2, 16
NW = NC * NS
NVEC = CHUNK // 16
RC_ROWS = CHUNK * 2 // 16
RP_ROWS = CHUNK * 3 // 16


def _sc_body(et_hbm, rid_hbm, rc_hbm, rp_hbm, wcomb_hbm, wlin_hbm, out_hbm,
             wlin_v, et_v, rid_v, cidx_v, rc_v, rp_v, rows_v, sem):
    c = lax.axis_index("c")
    s = lax.axis_index("s")
    wid = s * NC + c

    pltpu.sync_copy(wlin_hbm, wlin_v)
    ws = [[wlin_v[k, pl.ds(j * 16, 16)] for j in range(HID // 16)]
          for k in range(5)]

    nw_chunks = (NCHUNKS - wid + NW - 1) // NW

    def chunk_body(i, carry):
        cid = wid + NW * i
        base = cid * CHUNK
        pltpu.sync_copy(et_hbm.at[pl.ds(base, CHUNK)], et_v)
        pltpu.sync_copy(rid_hbm.at[pl.ds(base, CHUNK)], rid_v)
        pltpu.sync_copy(rc_hbm.at[pl.ds(cid * RC_ROWS, RC_ROWS)], rc_v)
        pltpu.sync_copy(rp_hbm.at[pl.ds(cid * RP_ROWS, RP_ROWS)], rp_v)

        for j in range(NVEC):
            sl = pl.ds(j * 16, 16)
            cidx_v[sl] = et_v[sl] * 9 + rid_v[sl]

        pltpu.async_copy(wcomb_hbm.at[cidx_v], rows_v, sem).wait()

        def group_body(g, carry2):
            vrc = [rc_v[g * 2 + t] for t in range(2)]
            vrp = [rp_v[g * 3 + t] for t in range(3)]
            for q in range(16):
                e = g * 16 + q
                f0 = 2 * q
                a0 = vrc[f0 // 16][f0 % 16]
                a1 = vrc[(f0 + 1) // 16][(f0 + 1) % 16]
                f1 = 3 * q
                p0 = vrp[f1 // 16][f1 % 16]
                p1 = vrp[(f1 + 1) // 16][(f1 + 1) % 16]
                p2 = vrp[(f1 + 2) // 16][(f1 + 2) % 16]
                for j in range(HID // 16):
                    sl = pl.ds(j * 16, 16)
                    v = rows_v[e, sl]
                    v = v + a0 * ws[0][j] + a1 * ws[1][j]
                    v = v + p0 * ws[2][j] + p1 * ws[3][j] + p2 * ws[4][j]
                    rows_v[e, sl] = v
            return carry2

        lax.fori_loop(0, CHUNK // 16, group_body, 0)

        pltpu.sync_copy(rows_v, out_hbm.at[pl.ds(base, CHUNK)])
        return carry

    lax.fori_loop(0, nw_chunks, chunk_body, 0)


@jax.jit
def _run(et, rid, rc, rp, wcomb, wlin):
    mesh = plsc.VectorSubcoreMesh(core_axis_name="c", subcore_axis_name="s",
                                  num_cores=NC, num_subcores=NS)
    fn = pl.kernel(
        _sc_body,
        out_type=jax.ShapeDtypeStruct((E, HID), jnp.float32),
        mesh=mesh,
        scratch_types=[
            pltpu.VMEM((5, HID), jnp.float32),
            pltpu.VMEM((CHUNK,), jnp.int32),
            pltpu.VMEM((CHUNK,), jnp.int32),
            pltpu.VMEM((CHUNK,), jnp.int32),
            pltpu.VMEM((RC_ROWS, 16), jnp.float32),
            pltpu.VMEM((RP_ROWS, 16), jnp.float32),
            pltpu.VMEM((CHUNK, HID), jnp.float32),
            pltpu.SemaphoreType.DMA,
        ],
    )
    return fn(et, rid, rc, rp, wcomb, wlin)


def kernel(etype, rid, att_rc, att_rp, W_type, W_rid, W_rc, b_rc, W_rp, b_rp):
    wcomb = (W_type[:, None, :] + W_rid[None, :, :]).reshape(-1, HID)
    wcomb = wcomb + b_rc + b_rp
    wlin = jnp.concatenate([W_rc, W_rp], axis=1).T
    rc16 = att_rc.reshape(E * 2 // 16, 16)
    rp16 = att_rp.reshape(E * 3 // 16, 16)
    return _run(etype.astype(jnp.int32), rid.astype(jnp.int32),
                rc16, rp16,
                wcomb.astype(jnp.float32), wlin.astype(jnp.float32))


# speedup vs baseline: 2.4473x; 1.5996x over previous
import jax
import jax.numpy as jnp
from jax import lax
from jax.experimental import pallas as pl
from jax.experimental.pallas import tpu as pltpu
from jax.experimental.pallas import tpu_sc as plsc

E = 320000
HID = 128
CHUNK = 128
NCHUNKS = E // CHUNK
NC, NS = 2, 16
NW = NC * NS
NVEC = CHUNK // 16
RC_ROWS = CHUNK * 2 // 16
RP_ROWS = CHUNK * 3 // 16


def _sc_body(et_hbm, rid_hbm, rc_hbm, rp_hbm, wcomb_hbm, wlin_hbm, out_hbm,
             wlin_v, et2, rid2, cidx2, rc2, rp2, rows2,
             isem0, isem1, gsem0, gsem1, osem0, osem1):
    isem = (isem0, isem1)
    gsem = (gsem0, gsem1)
    osem = (osem0, osem1)
    c = lax.axis_index("c")
    s = lax.axis_index("s")
    wid = s * NC + c

    pltpu.sync_copy(wlin_hbm, wlin_v)
    ws = [[wlin_v[k, pl.ds(j * 16, 16)] for j in range(HID // 16)]
          for k in range(5)]

    n = (NCHUNKS - wid + NW - 1) // NW

    def in_slices(i, b):
        cid = wid + NW * i
        return ((et_hbm.at[pl.ds(cid * CHUNK, CHUNK)], et2.at[b]),
                (rid_hbm.at[pl.ds(cid * CHUNK, CHUNK)], rid2.at[b]),
                (rc_hbm.at[pl.ds(cid * RC_ROWS, RC_ROWS)], rc2.at[b]),
                (rp_hbm.at[pl.ds(cid * RP_ROWS, RP_ROWS)], rp2.at[b]))

    def issue_inputs(i, b):
        for src, dst in in_slices(i, b):
            pltpu.async_copy(src, dst, isem[b])

    def wait_inputs(b):
        for src, dst in in_slices(0, b):
            pltpu.make_async_copy(src, dst, isem[b]).wait()

    def compute_cidx(b):
        for j in range(NVEC):
            sl = pl.ds(j * 16, 16)
            cidx2[b, sl] = et2[b, sl] * 9 + rid2[b, sl]

    def issue_gather(b):
        pltpu.async_copy(wcomb_hbm.at[cidx2.at[b]], rows2.at[b], gsem[b])

    def wait_gather(b):
        pltpu.make_async_copy(out_hbm.at[pl.ds(0, CHUNK)], rows2.at[b],
                              gsem[b]).wait()

    def issue_out(i, b):
        cid = wid + NW * i
        pltpu.async_copy(rows2.at[b], out_hbm.at[pl.ds(cid * CHUNK, CHUNK)],
                         osem[b])

    def wait_out(b):
        pltpu.make_async_copy(rows2.at[b], out_hbm.at[pl.ds(0, CHUNK)],
                              osem[b]).wait()

    def fma(b):
        def group_body(g, carry):
            vrc = [rc2[b, g * 2 + t] for t in range(2)]
            vrp = [rp2[b, g * 3 + t] for t in range(3)]
            for q in range(16):
                e = g * 16 + q
                f0 = 2 * q
                a0 = vrc[f0 // 16][f0 % 16]
                a1 = vrc[(f0 + 1) // 16][(f0 + 1) % 16]
                f1 = 3 * q
                p0 = vrp[f1 // 16][f1 % 16]
                p1 = vrp[(f1 + 1) // 16][(f1 + 1) % 16]
                p2 = vrp[(f1 + 2) // 16][(f1 + 2) % 16]
                for j in range(HID // 16):
                    sl = pl.ds(j * 16, 16)
                    v = rows2[b, e, sl]
                    v = v + a0 * ws[0][j] + a1 * ws[1][j]
                    v = v + p0 * ws[2][j] + p1 * ws[3][j] + p2 * ws[4][j]
                    rows2[b, e, sl] = v
            return carry

        lax.fori_loop(0, CHUNK // 16, group_body, 0)

    def process(i, b):
        @pl.when(i + 1 < n)
        def _():
            wait_inputs(1 - b)
            compute_cidx(1 - b)

        @pl.when(i >= 1)
        def _():
            wait_out(1 - b)

        @pl.when(i + 1 < n)
        def _():
            issue_gather(1 - b)

        wait_gather(b)
        fma(b)
        issue_out(i, b)

        @pl.when(i + 2 < n)
        def _():
            issue_inputs(i + 2, b)

    for src, dst in in_slices(0, 0):
        pltpu.sync_copy(src, dst)
    compute_cidx(0)
    issue_gather(0)
    issue_inputs(1, 1)

    def pair_body(p, carry):
        process(2 * p, 0)
        process(2 * p + 1, 1)
        return carry

    lax.fori_loop(0, n // 2, pair_body, 0)

    @pl.when(n % 2 == 1)
    def _():
        process(n - 1, 0)

    @pl.when(n % 2 == 1)
    def _():
        wait_out(0)

    @pl.when(n % 2 == 0)
    def _():
        wait_out(1)


@jax.jit
def _run(et, rid, rc, rp, wcomb, wlin):
    mesh = plsc.VectorSubcoreMesh(core_axis_name="c", subcore_axis_name="s",
                                  num_cores=NC, num_subcores=NS)
    fn = pl.kernel(
        _sc_body,
        out_type=jax.ShapeDtypeStruct((E, HID), jnp.float32),
        mesh=mesh,
        scratch_types=[
            pltpu.VMEM((5, HID), jnp.float32),
            pltpu.VMEM((2, CHUNK), jnp.int32),
            pltpu.VMEM((2, CHUNK), jnp.int32),
            pltpu.VMEM((2, CHUNK), jnp.int32),
            pltpu.VMEM((2, RC_ROWS, 16), jnp.float32),
            pltpu.VMEM((2, RP_ROWS, 16), jnp.float32),
            pltpu.VMEM((2, CHUNK, HID), jnp.float32),
            pltpu.SemaphoreType.DMA,
            pltpu.SemaphoreType.DMA,
            pltpu.SemaphoreType.DMA,
            pltpu.SemaphoreType.DMA,
            pltpu.SemaphoreType.DMA,
            pltpu.SemaphoreType.DMA,
        ],
    )
    return fn(et, rid, rc, rp, wcomb, wlin)


def kernel(etype, rid, att_rc, att_rp, W_type, W_rid, W_rc, b_rc, W_rp, b_rp):
    wcomb = (W_type[:, None, :] + W_rid[None, :, :]).reshape(-1, HID)
    wcomb = wcomb + b_rc + b_rp
    wlin = jnp.concatenate([W_rc, W_rp], axis=1).T
    rc16 = att_rc.reshape(E * 2 // 16, 16)
    rp16 = att_rp.reshape(E * 3 // 16, 16)
    return _run(etype.astype(jnp.int32), rid.astype(jnp.int32),
                rc16, rp16,
                wcomb.astype(jnp.float32), wlin.astype(jnp.float32))


# speedup vs baseline: 2.6964x; 1.1018x over previous
import jax
import jax.numpy as jnp
from jax import lax
from jax.experimental import pallas as pl
from jax.experimental.pallas import tpu as pltpu
from jax.experimental.pallas import tpu_sc as plsc

E = 320000
HID = 128
CHUNK = 128
NCHUNKS = E // CHUNK
NC, NS = 2, 16
NW = NC * NS
NVEC = CHUNK // 16

BE = 2000


def _tc_body(rc_ref, rp_ref, wrc_ref, wrp_ref, out_ref):
    acc = jnp.dot(rc_ref[...], wrc_ref[...],
                  preferred_element_type=jnp.float32)
    acc = acc + jnp.dot(rp_ref[...], wrp_ref[...],
                        preferred_element_type=jnp.float32)
    out_ref[...] = acc


def _sc_body(et_hbm, rid_hbm, a_hbm, wcomb_hbm, out_hbm,
             et2, rid2, cidx2, a2, rows2,
             isem0, isem1, gsem0, gsem1, osem0, osem1):
    isem = (isem0, isem1)
    gsem = (gsem0, gsem1)
    osem = (osem0, osem1)
    c = lax.axis_index("c")
    s = lax.axis_index("s")
    wid = s * NC + c

    n = (NCHUNKS - wid + NW - 1) // NW

    def in_slices(i, b):
        cid = wid + NW * i
        return ((et_hbm.at[pl.ds(cid * CHUNK, CHUNK)], et2.at[b]),
                (rid_hbm.at[pl.ds(cid * CHUNK, CHUNK)], rid2.at[b]),
                (a_hbm.at[pl.ds(cid * CHUNK, CHUNK)], a2.at[b]))

    def issue_inputs(i, b):
        for src, dst in in_slices(i, b):
            pltpu.async_copy(src, dst, isem[b])

    def wait_inputs(b):
        for src, dst in in_slices(0, b):
            pltpu.make_async_copy(src, dst, isem[b]).wait()

    def compute_cidx(b):
        for j in range(NVEC):
            sl = pl.ds(j * 16, 16)
            cidx2[b, sl] = et2[b, sl] * 9 + rid2[b, sl]

    def issue_gather(b):
        pltpu.async_copy(wcomb_hbm.at[cidx2.at[b]], rows2.at[b], gsem[b])

    def wait_gather(b):
        pltpu.make_async_copy(out_hbm.at[pl.ds(0, CHUNK)], rows2.at[b],
                              gsem[b]).wait()

    def issue_out(i, b):
        cid = wid + NW * i
        pltpu.async_copy(rows2.at[b], out_hbm.at[pl.ds(cid * CHUNK, CHUNK)],
                         osem[b])

    def wait_out(b):
        pltpu.make_async_copy(rows2.at[b], out_hbm.at[pl.ds(0, CHUNK)],
                              osem[b]).wait()

    def merge(b):
        def edge_body(e2, carry):
            for u in range(2):
                e = e2 * 2 + u
                for j in range(HID // 16):
                    sl = pl.ds(j * 16, 16)
                    plsc.addupdate(rows2.at[b, e, sl], a2[b, e, sl])
            return carry

        lax.fori_loop(0, CHUNK // 2, edge_body, 0)

    def process(i, b):
        @pl.when(i + 1 < n)
        def _():
            wait_inputs(1 - b)
            compute_cidx(1 - b)

        @pl.when(i >= 1)
        def _():
            wait_out(1 - b)

        @pl.when(i + 1 < n)
        def _():
            issue_gather(1 - b)

        wait_gather(b)
        merge(b)
        issue_out(i, b)

        @pl.when(i + 2 < n)
        def _():
            issue_inputs(i + 2, b)

    for src, dst in in_slices(0, 0):
        pltpu.sync_copy(src, dst)
    compute_cidx(0)
    issue_gather(0)
    issue_inputs(1, 1)

    def pair_body(p, carry):
        process(2 * p, 0)
        process(2 * p + 1, 1)
        return carry

    lax.fori_loop(0, n // 2, pair_body, 0)

    @pl.when(n % 2 == 1)
    def _():
        process(n - 1, 0)

    @pl.when(n % 2 == 1)
    def _():
        wait_out(0)

    @pl.when(n % 2 == 0)
    def _():
        wait_out(1)


@jax.jit
def _run(et, rid, rc, rp, wcomb, wrc_t, wrp_t):
    a = pl.pallas_call(
        _tc_body,
        grid=(E // BE,),
        in_specs=[
            pl.BlockSpec((BE, 2), lambda i: (i, 0)),
            pl.BlockSpec((BE, 3), lambda i: (i, 0)),
            pl.BlockSpec((2, HID), lambda i: (0, 0)),
            pl.BlockSpec((3, HID), lambda i: (0, 0)),
        ],
        out_specs=pl.BlockSpec((BE, HID), lambda i: (i, 0)),
        out_shape=jax.ShapeDtypeStruct((E, HID), jnp.float32),
    )(rc, rp, wrc_t, wrp_t)

    mesh = plsc.VectorSubcoreMesh(core_axis_name="c", subcore_axis_name="s",
                                  num_cores=NC, num_subcores=NS)
    fn = pl.kernel(
        _sc_body,
        out_type=jax.ShapeDtypeStruct((E, HID), jnp.float32),
        mesh=mesh,
        scratch_types=[
            pltpu.VMEM((2, CHUNK), jnp.int32),
            pltpu.VMEM((2, CHUNK), jnp.int32),
            pltpu.VMEM((2, CHUNK), jnp.int32),
            pltpu.VMEM((2, CHUNK, HID), jnp.float32),
            pltpu.VMEM((2, CHUNK, HID), jnp.float32),
            pltpu.SemaphoreType.DMA,
            pltpu.SemaphoreType.DMA,
            pltpu.SemaphoreType.DMA,
            pltpu.SemaphoreType.DMA,
            pltpu.SemaphoreType.DMA,
            pltpu.SemaphoreType.DMA,
        ],
    )
    return fn(et, rid, a, wcomb)


def kernel(etype, rid, att_rc, att_rp, W_type, W_rid, W_rc, b_rc, W_rp, b_rp):
    wcomb = (W_type[:, None, :] + W_rid[None, :, :]).reshape(-1, HID)
    wcomb = wcomb + b_rc + b_rp
    return _run(etype.astype(jnp.int32), rid.astype(jnp.int32),
                att_rc, att_rp, wcomb.astype(jnp.float32),
                W_rc.T.astype(jnp.float32), W_rp.T.astype(jnp.float32))


# speedup vs baseline: 2.9193x; 1.0827x over previous
import jax
import jax.numpy as jnp
from jax import lax
from jax.experimental import pallas as pl
from jax.experimental.pallas import tpu as pltpu
from jax.experimental.pallas import tpu_sc as plsc

E = 320000
HID = 128
CHUNK = 128
NCHUNKS = E // CHUNK
NC, NS = 2, 16
NW = NC * NS
NVEC = CHUNK // 16
NTAB = 35 * 9

BE = 2000


def _tc_body(rc_ref, rp_ref, wrc_ref, wrp_ref, out_ref):
    acc = jnp.dot(rc_ref[...], wrc_ref[...],
                  preferred_element_type=jnp.float32)
    acc = acc + jnp.dot(rp_ref[...], wrp_ref[...],
                        preferred_element_type=jnp.float32)
    out_ref[...] = acc


def _sc_body(et_hbm, rid_hbm, a_hbm, wcomb_hbm, out_hbm,
             et2, rid2, cidx2, a2, rows2,
             isem0, isem1, gsem0, gsem1, osem0, osem1):
    isem = (isem0, isem1)
    gsem = (gsem0, gsem1)
    osem = (osem0, osem1)
    c = lax.axis_index("c")
    s = lax.axis_index("s")
    wid = s * NC + c

    n = (NCHUNKS - wid + NW - 1) // NW

    def in_slices(i, b):
        cid = wid + NW * i
        return ((et_hbm.at[pl.ds(cid * CHUNK, CHUNK)], et2.at[b]),
                (rid_hbm.at[pl.ds(cid * CHUNK, CHUNK)], rid2.at[b]),
                (a_hbm.at[pl.ds(cid * CHUNK, CHUNK)], a2.at[b]))

    def issue_inputs(i, b):
        for src, dst in in_slices(i, b):
            pltpu.async_copy(src, dst, isem[b])

    def wait_inputs(b):
        for src, dst in in_slices(0, b):
            pltpu.make_async_copy(src, dst, isem[b]).wait()

    tab_base = wid * NTAB

    def compute_cidx(b):
        for j in range(NVEC):
            sl = pl.ds(j * 16, 16)
            cidx2[b, sl] = et2[b, sl] * 9 + rid2[b, sl] + tab_base

    def issue_gather(b):
        pltpu.async_copy(wcomb_hbm.at[cidx2.at[b]], rows2.at[b], gsem[b])

    def wait_gather(b):
        pltpu.make_async_copy(out_hbm.at[pl.ds(0, CHUNK)], rows2.at[b],
                              gsem[b]).wait()

    def issue_out(i, b):
        cid = wid + NW * i
        pltpu.async_copy(rows2.at[b], out_hbm.at[pl.ds(cid * CHUNK, CHUNK)],
                         osem[b])

    def wait_out(b):
        pltpu.make_async_copy(rows2.at[b], out_hbm.at[pl.ds(0, CHUNK)],
                              osem[b]).wait()

    def merge(b):
        def edge_body(e2, carry):
            for u in range(2):
                e = e2 * 2 + u
                for j in range(HID // 16):
                    sl = pl.ds(j * 16, 16)
                    plsc.addupdate(rows2.at[b, e, sl], a2[b, e, sl])
            return carry

        lax.fori_loop(0, CHUNK // 2, edge_body, 0)

    def process(i, b):
        @pl.when(i + 1 < n)
        def _():
            wait_inputs(1 - b)
            compute_cidx(1 - b)

        @pl.when(i >= 1)
        def _():
            wait_out(1 - b)

        @pl.when(i + 1 < n)
        def _():
            issue_gather(1 - b)

        wait_gather(b)
        merge(b)
        issue_out(i, b)

        @pl.when(i + 2 < n)
        def _():
            issue_inputs(i + 2, b)

    for src, dst in in_slices(0, 0):
        pltpu.sync_copy(src, dst)
    compute_cidx(0)
    issue_gather(0)
    issue_inputs(1, 1)

    def pair_body(p, carry):
        process(2 * p, 0)
        process(2 * p + 1, 1)
        return carry

    lax.fori_loop(0, n // 2, pair_body, 0)

    @pl.when(n % 2 == 1)
    def _():
        process(n - 1, 0)

    @pl.when(n % 2 == 1)
    def _():
        wait_out(0)

    @pl.when(n % 2 == 0)
    def _():
        wait_out(1)


@jax.jit
def _run(et, rid, rc, rp, wcomb, wrc_t, wrp_t):
    a = pl.pallas_call(
        _tc_body,
        grid=(E // BE,),
        in_specs=[
            pl.BlockSpec((BE, 2), lambda i: (i, 0)),
            pl.BlockSpec((BE, 3), lambda i: (i, 0)),
            pl.BlockSpec((2, HID), lambda i: (0, 0)),
            pl.BlockSpec((3, HID), lambda i: (0, 0)),
        ],
        out_specs=pl.BlockSpec((BE, HID), lambda i: (i, 0)),
        out_shape=jax.ShapeDtypeStruct((E, HID), jnp.float32),
    )(rc, rp, wrc_t, wrp_t)

    mesh = plsc.VectorSubcoreMesh(core_axis_name="c", subcore_axis_name="s",
                                  num_cores=NC, num_subcores=NS)
    fn = pl.kernel(
        _sc_body,
        out_type=jax.ShapeDtypeStruct((E, HID), jnp.float32),
        mesh=mesh,
        scratch_types=[
            pltpu.VMEM((2, CHUNK), jnp.int32),
            pltpu.VMEM((2, CHUNK), jnp.int32),
            pltpu.VMEM((2, CHUNK), jnp.int32),
            pltpu.VMEM((2, CHUNK, HID), jnp.float32),
            pltpu.VMEM((2, CHUNK, HID), jnp.float32),
            pltpu.SemaphoreType.DMA,
            pltpu.SemaphoreType.DMA,
            pltpu.SemaphoreType.DMA,
            pltpu.SemaphoreType.DMA,
            pltpu.SemaphoreType.DMA,
            pltpu.SemaphoreType.DMA,
        ],
    )
    return fn(et, rid, a, wcomb)


def kernel(etype, rid, att_rc, att_rp, W_type, W_rid, W_rc, b_rc, W_rp, b_rp):
    wcomb = (W_type[:, None, :] + W_rid[None, :, :]).reshape(-1, HID)
    wcomb = wcomb + b_rc + b_rp
    wcomb_rep = jnp.tile(wcomb, (NW, 1))
    return _run(etype.astype(jnp.int32), rid.astype(jnp.int32),
                att_rc, att_rp, wcomb_rep.astype(jnp.float32),
                W_rc.T.astype(jnp.float32), W_rp.T.astype(jnp.float32))


# speedup vs baseline: 3.1945x; 1.0942x over previous
import jax
import jax.numpy as jnp
from jax import lax
from jax.experimental import pallas as pl
from jax.experimental.pallas import tpu as pltpu
from jax.experimental.pallas import tpu_sc as plsc

E = 320000
HID = 128
CHUNK = 128
NCHUNKS = E // CHUNK
NC, NS = 2, 16
NW = NC * NS
NVEC = CHUNK // 16
NTAB = 35 * 9

BE = 2000


def _tc_body(rc_ref, rp_ref, wrc_ref, wrp_ref, out_ref):
    acc = jnp.dot(rc_ref[...], wrc_ref[...],
                  preferred_element_type=jnp.float32)
    acc = acc + jnp.dot(rp_ref[...], wrp_ref[...],
                        preferred_element_type=jnp.float32)
    out_ref[...] = acc


def _sc_body(et_hbm, rid_hbm, a_hbm, wcomb_hbm, out_hbm,
             et2, rid2, cidx2, rows2,
             isem0, isem1, gsem0, gsem1, osem0, osem1):
    isem = (isem0, isem1)
    gsem = (gsem0, gsem1)
    osem = (osem0, osem1)
    c = lax.axis_index("c")
    s = lax.axis_index("s")
    wid = s * NC + c

    n = (NCHUNKS - wid + NW - 1) // NW

    def in_slices(i, b):
        cid = wid + NW * i
        return ((et_hbm.at[pl.ds(cid * CHUNK, CHUNK)], et2.at[b]),
                (rid_hbm.at[pl.ds(cid * CHUNK, CHUNK)], rid2.at[b]),
                (a_hbm.at[pl.ds(cid * CHUNK, CHUNK)], rows2.at[b]))

    def issue_inputs(i, b):
        for src, dst in in_slices(i, b):
            pltpu.async_copy(src, dst, isem[b])

    def wait_inputs(b):
        for src, dst in in_slices(0, b):
            pltpu.make_async_copy(src, dst, isem[b]).wait()

    tab_base = wid * NTAB

    def compute_cidx(b):
        for j in range(NVEC):
            sl = pl.ds(j * 16, 16)
            cidx2[b, sl] = et2[b, sl] * 9 + rid2[b, sl] + tab_base

    def issue_gather(b):
        pltpu.async_copy(wcomb_hbm.at[cidx2.at[b]], rows2.at[b], gsem[b],
                         add=True)

    def wait_gather(b):
        pltpu.make_async_copy(out_hbm.at[pl.ds(0, CHUNK)], rows2.at[b],
                              gsem[b]).wait()

    def issue_out(i, b):
        cid = wid + NW * i
        pltpu.async_copy(rows2.at[b], out_hbm.at[pl.ds(cid * CHUNK, CHUNK)],
                         osem[b])

    def wait_out(b):
        pltpu.make_async_copy(rows2.at[b], out_hbm.at[pl.ds(0, CHUNK)],
                              osem[b]).wait()

    def process(i, b):
        @pl.when(i >= 1)
        def _():
            wait_out(1 - b)

        @pl.when(jnp.logical_and(i >= 1, i + 1 < n))
        def _():
            issue_inputs(i + 1, 1 - b)

        wait_gather(b)
        issue_out(i, b)

        @pl.when(i + 1 < n)
        def _():
            wait_inputs(1 - b)
            compute_cidx(1 - b)
            issue_gather(1 - b)

    for src, dst in in_slices(0, 0):
        pltpu.sync_copy(src, dst)
    compute_cidx(0)
    issue_gather(0)
    issue_inputs(1, 1)

    def pair_body(p, carry):
        process(2 * p, 0)
        process(2 * p + 1, 1)
        return carry

    lax.fori_loop(0, n // 2, pair_body, 0)

    @pl.when(n % 2 == 1)
    def _():
        process(n - 1, 0)

    @pl.when(n % 2 == 1)
    def _():
        wait_out(0)

    @pl.when(n % 2 == 0)
    def _():
        wait_out(1)


@jax.jit
def _run(et, rid, rc, rp, wcomb, wrc_t, wrp_t):
    a = pl.pallas_call(
        _tc_body,
        grid=(E // BE,),
        in_specs=[
            pl.BlockSpec((BE, 2), lambda i: (i, 0)),
            pl.BlockSpec((BE, 3), lambda i: (i, 0)),
            pl.BlockSpec((2, HID), lambda i: (0, 0)),
            pl.BlockSpec((3, HID), lambda i: (0, 0)),
        ],
        out_specs=pl.BlockSpec((BE, HID), lambda i: (i, 0)),
        out_shape=jax.ShapeDtypeStruct((E, HID), jnp.float32),
    )(rc, rp, wrc_t, wrp_t)

    mesh = plsc.VectorSubcoreMesh(core_axis_name="c", subcore_axis_name="s",
                                  num_cores=NC, num_subcores=NS)
    fn = pl.kernel(
        _sc_body,
        out_type=jax.ShapeDtypeStruct((E, HID), jnp.float32),
        mesh=mesh,
        scratch_types=[
            pltpu.VMEM((2, CHUNK), jnp.int32),
            pltpu.VMEM((2, CHUNK), jnp.int32),
            pltpu.VMEM((2, CHUNK), jnp.int32),
            pltpu.VMEM((2, CHUNK, HID), jnp.float32),
            pltpu.SemaphoreType.DMA,
            pltpu.SemaphoreType.DMA,
            pltpu.SemaphoreType.DMA,
            pltpu.SemaphoreType.DMA,
            pltpu.SemaphoreType.DMA,
            pltpu.SemaphoreType.DMA,
        ],
    )
    return fn(et, rid, a, wcomb)


def kernel(etype, rid, att_rc, att_rp, W_type, W_rid, W_rc, b_rc, W_rp, b_rp):
    wcomb = (W_type[:, None, :] + W_rid[None, :, :]).reshape(-1, HID)
    wcomb = wcomb + b_rc + b_rp
    wcomb_rep = jnp.tile(wcomb, (NW, 1))
    return _run(etype.astype(jnp.int32), rid.astype(jnp.int32),
                att_rc, att_rp, wcomb_rep.astype(jnp.float32),
                W_rc.T.astype(jnp.float32), W_rp.T.astype(jnp.float32))


# speedup vs baseline: 3.2318x; 1.0117x over previous
import jax
import jax.numpy as jnp
from jax import lax
from jax.experimental import pallas as pl
from jax.experimental.pallas import tpu as pltpu
from jax.experimental.pallas import tpu_sc as plsc

E = 320000
HID = 128
CHUNK = 128
NCHUNKS = E // CHUNK
NC, NS = 2, 16
NW = NC * NS
NVEC = CHUNK // 16
NTAB = 35 * 9
NBUF = 3

BE = 2000


def _tc_body(rc_ref, rp_ref, wrc_ref, wrp_ref, out_ref):
    acc = jnp.dot(rc_ref[...], wrc_ref[...],
                  preferred_element_type=jnp.float32)
    acc = acc + jnp.dot(rp_ref[...], wrp_ref[...],
                        preferred_element_type=jnp.float32)
    out_ref[...] = acc


def _sc_body(et_hbm, rid_hbm, a_hbm, wcomb_hbm, out_hbm,
             et3, rid3, cidx3, rows3,
             isem0, isem1, isem2, gsem0, gsem1, gsem2,
             osem0, osem1, osem2):
    isem = (isem0, isem1, isem2)
    gsem = (gsem0, gsem1, gsem2)
    osem = (osem0, osem1, osem2)
    c = lax.axis_index("c")
    s = lax.axis_index("s")
    wid = s * NC + c

    n = (NCHUNKS - wid + NW - 1) // NW

    tab_base = wid * NTAB

    def in_slices(i, b):
        cid = wid + NW * i
        return ((et_hbm.at[pl.ds(cid * CHUNK, CHUNK)], et3.at[b]),
                (rid_hbm.at[pl.ds(cid * CHUNK, CHUNK)], rid3.at[b]),
                (a_hbm.at[pl.ds(cid * CHUNK, CHUNK)], rows3.at[b]))

    def issue_inputs(i, b):
        for src, dst in in_slices(i, b):
            pltpu.async_copy(src, dst, isem[b])

    def wait_inputs(b):
        for src, dst in in_slices(0, b):
            pltpu.make_async_copy(src, dst, isem[b]).wait()

    def compute_cidx(b):
        for j in range(NVEC):
            sl = pl.ds(j * 16, 16)
            cidx3[b, sl] = et3[b, sl] * 9 + rid3[b, sl] + tab_base

    def issue_gather(b):
        pltpu.async_copy(wcomb_hbm.at[cidx3.at[b]], rows3.at[b], gsem[b],
                         add=True)

    def wait_gather(b):
        pltpu.make_async_copy(out_hbm.at[pl.ds(0, CHUNK)], rows3.at[b],
                              gsem[b]).wait()

    def issue_out(i, b):
        cid = wid + NW * i
        pltpu.async_copy(rows3.at[b], out_hbm.at[pl.ds(cid * CHUNK, CHUNK)],
                         osem[b])

    def wait_out(b):
        pltpu.make_async_copy(rows3.at[b], out_hbm.at[pl.ds(0, CHUNK)],
                              osem[b]).wait()

    def process(i, b):
        bn = (b + 1) % NBUF
        bp = (b + 2) % NBUF

        @pl.when(i >= 1)
        def _():
            wait_out(bp)

        @pl.when(jnp.logical_and(i >= 1, i + 2 < n))
        def _():
            issue_inputs(i + 2, bp)

        wait_gather(b)
        issue_out(i, b)

        @pl.when(i + 1 < n)
        def _():
            wait_inputs(bn)
            compute_cidx(bn)
            issue_gather(bn)

    for src, dst in in_slices(0, 0):
        pltpu.sync_copy(src, dst)
    compute_cidx(0)
    issue_gather(0)
    issue_inputs(1, 1)
    issue_inputs(2, 2)

    def trip_body(p, carry):
        process(3 * p, 0)
        process(3 * p + 1, 1)
        process(3 * p + 2, 2)
        return carry

    lax.fori_loop(0, n // NBUF, trip_body, 0)

    @pl.when(n % NBUF == 1)
    def _():
        process(n - 1, 0)

    @pl.when(n % NBUF == 0)
    def _():
        wait_out(2)

    @pl.when(n % NBUF == 1)
    def _():
        wait_out(0)


@jax.jit
def _run(et, rid, rc, rp, wcomb, wrc_t, wrp_t):
    a = pl.pallas_call(
        _tc_body,
        grid=(E // BE,),
        in_specs=[
            pl.BlockSpec((BE, 2), lambda i: (i, 0)),
            pl.BlockSpec((BE, 3), lambda i: (i, 0)),
            pl.BlockSpec((2, HID), lambda i: (0, 0)),
            pl.BlockSpec((3, HID), lambda i: (0, 0)),
        ],
        out_specs=pl.BlockSpec((BE, HID), lambda i: (i, 0)),
        out_shape=jax.ShapeDtypeStruct((E, HID), jnp.float32),
    )(rc, rp, wrc_t, wrp_t)

    mesh = plsc.VectorSubcoreMesh(core_axis_name="c", subcore_axis_name="s",
                                  num_cores=NC, num_subcores=NS)
    fn = pl.kernel(
        _sc_body,
        out_type=jax.ShapeDtypeStruct((E, HID), jnp.float32),
        mesh=mesh,
        scratch_types=[
            pltpu.VMEM((NBUF, CHUNK), jnp.int32),
            pltpu.VMEM((NBUF, CHUNK), jnp.int32),
            pltpu.VMEM((NBUF, CHUNK), jnp.int32),
            pltpu.VMEM((NBUF, CHUNK, HID), jnp.float32),
            pltpu.SemaphoreType.DMA,
            pltpu.SemaphoreType.DMA,
            pltpu.SemaphoreType.DMA,
            pltpu.SemaphoreType.DMA,
            pltpu.SemaphoreType.DMA,
            pltpu.SemaphoreType.DMA,
            pltpu.SemaphoreType.DMA,
            pltpu.SemaphoreType.DMA,
            pltpu.SemaphoreType.DMA,
        ],
    )
    return fn(et, rid, a, wcomb)


def kernel(etype, rid, att_rc, att_rp, W_type, W_rid, W_rc, b_rc, W_rp, b_rp):
    wcomb = (W_type[:, None, :] + W_rid[None, :, :]).reshape(-1, HID)
    wcomb = wcomb + b_rc + b_rp
    wcomb_rep = jnp.tile(wcomb, (NW, 1))
    return _run(etype.astype(jnp.int32), rid.astype(jnp.int32),
                att_rc, att_rp, wcomb_rep.astype(jnp.float32),
                W_rc.T.astype(jnp.float32), W_rp.T.astype(jnp.float32))


# speedup vs baseline: 3.5102x; 1.0862x over previous
import jax
import jax.numpy as jnp
from jax import lax
from jax.experimental import pallas as pl
from jax.experimental.pallas import tpu as pltpu
from jax.experimental.pallas import tpu_sc as plsc

E = 320000
HID = 128
CHUNK = 256
GATH = 128
NCHUNKS = E // CHUNK
NC, NS = 2, 16
NW = NC * NS
NTAB = 35 * 9
NBUF = 3

BE = 4000


def _tc_body(rc_ref, rp_ref, wrc_ref, wrp_ref, out_ref):
    acc = jnp.dot(rc_ref[...], wrc_ref[...],
                  preferred_element_type=jnp.float32)
    acc = acc + jnp.dot(rp_ref[...], wrp_ref[...],
                        preferred_element_type=jnp.float32)
    out_ref[...] = acc


def _sc_body(et_hbm, rid_hbm, a_hbm, wcomb_hbm, out_hbm,
             et3, rid3, cidx3, rows3,
             isem0, isem1, isem2, gsem0, gsem1, gsem2,
             osem0, osem1, osem2):
    isem = (isem0, isem1, isem2)
    gsem = (gsem0, gsem1, gsem2)
    osem = (osem0, osem1, osem2)
    c = lax.axis_index("c")
    s = lax.axis_index("s")
    wid = s * NC + c

    n = (NCHUNKS - wid + NW - 1) // NW

    tab_base = wid * NTAB

    NG = CHUNK // GATH

    def in_slices(i, b):
        cid = wid + NW * i
        return ((et_hbm.at[pl.ds(cid * NG, NG)], et3.at[b]),
                (rid_hbm.at[pl.ds(cid * NG, NG)], rid3.at[b]),
                (a_hbm.at[pl.ds(cid * CHUNK, CHUNK)], rows3.at[b]))

    def issue_inputs(i, b):
        for src, dst in in_slices(i, b):
            pltpu.async_copy(src, dst, isem[b])

    def wait_inputs(b):
        for src, dst in in_slices(0, b):
            pltpu.make_async_copy(src, dst, isem[b]).wait()

    def compute_cidx(b):
        for h in range(CHUNK // GATH):
            for j in range(GATH // 16):
                sl = pl.ds(j * 16, 16)
                cidx3[b, h, sl] = et3[b, h, sl] * 9 + rid3[b, h, sl] + tab_base

    def issue_gather(b):
        for h in range(CHUNK // GATH):
            pltpu.async_copy(wcomb_hbm.at[cidx3.at[b, h]],
                             rows3.at[b, pl.ds(h * GATH, GATH)], gsem[b],
                             add=True)

    def wait_gather(b):
        pltpu.make_async_copy(out_hbm.at[pl.ds(0, CHUNK)], rows3.at[b],
                              gsem[b]).wait()

    def issue_out(i, b):
        cid = wid + NW * i
        pltpu.async_copy(rows3.at[b], out_hbm.at[pl.ds(cid * CHUNK, CHUNK)],
                         osem[b])

    def wait_out(b):
        pltpu.make_async_copy(rows3.at[b], out_hbm.at[pl.ds(0, CHUNK)],
                              osem[b]).wait()

    def process(i, b):
        bn = (b + 1) % NBUF
        bp = (b + 2) % NBUF

        @pl.when(i >= 1)
        def _():
            wait_out(bp)

        @pl.when(jnp.logical_and(i >= 1, i + 2 < n))
        def _():
            issue_inputs(i + 2, bp)

        wait_gather(b)
        issue_out(i, b)

        @pl.when(i + 1 < n)
        def _():
            wait_inputs(bn)
            compute_cidx(bn)
            issue_gather(bn)

    for src, dst in in_slices(0, 0):
        pltpu.sync_copy(src, dst)
    compute_cidx(0)
    issue_gather(0)
    issue_inputs(1, 1)
    issue_inputs(2, 2)

    def trip_body(p, carry):
        process(3 * p, 0)
        process(3 * p + 1, 1)
        process(3 * p + 2, 2)
        return carry

    lax.fori_loop(0, n // NBUF, trip_body, 0)

    @pl.when(n % NBUF == 1)
    def _():
        process(n - 1, 0)

    @pl.when(n % NBUF == 0)
    def _():
        wait_out(2)

    @pl.when(n % NBUF == 1)
    def _():
        wait_out(0)


@jax.jit
def _run(et, rid, rc, rp, wcomb, wrc_t, wrp_t):
    a = pl.pallas_call(
        _tc_body,
        grid=(E // BE,),
        in_specs=[
            pl.BlockSpec((BE, 2), lambda i: (i, 0)),
            pl.BlockSpec((BE, 3), lambda i: (i, 0)),
            pl.BlockSpec((2, HID), lambda i: (0, 0)),
            pl.BlockSpec((3, HID), lambda i: (0, 0)),
        ],
        out_specs=pl.BlockSpec((BE, HID), lambda i: (i, 0)),
        out_shape=jax.ShapeDtypeStruct((E, HID), jnp.float32),
    )(rc, rp, wrc_t, wrp_t)

    mesh = plsc.VectorSubcoreMesh(core_axis_name="c", subcore_axis_name="s",
                                  num_cores=NC, num_subcores=NS)
    fn = pl.kernel(
        _sc_body,
        out_type=jax.ShapeDtypeStruct((E, HID), jnp.float32),
        mesh=mesh,
        scratch_types=[
            pltpu.VMEM((NBUF, CHUNK // GATH, GATH), jnp.int32),
            pltpu.VMEM((NBUF, CHUNK // GATH, GATH), jnp.int32),
            pltpu.VMEM((NBUF, CHUNK // GATH, GATH), jnp.int32),
            pltpu.VMEM((NBUF, CHUNK, HID), jnp.float32),
            pltpu.SemaphoreType.DMA,
            pltpu.SemaphoreType.DMA,
            pltpu.SemaphoreType.DMA,
            pltpu.SemaphoreType.DMA,
            pltpu.SemaphoreType.DMA,
            pltpu.SemaphoreType.DMA,
            pltpu.SemaphoreType.DMA,
            pltpu.SemaphoreType.DMA,
            pltpu.SemaphoreType.DMA,
        ],
    )
    return fn(et, rid, a, wcomb)


def kernel(etype, rid, att_rc, att_rp, W_type, W_rid, W_rc, b_rc, W_rp, b_rp):
    wcomb = (W_type[:, None, :] + W_rid[None, :, :]).reshape(-1, HID)
    wcomb = wcomb + b_rc + b_rp
    wcomb_rep = jnp.tile(wcomb, (NW, 1))
    et128 = etype.astype(jnp.int32).reshape(E // GATH, GATH)
    rid128 = rid.astype(jnp.int32).reshape(E // GATH, GATH)
    return _run(et128, rid128,
                att_rc, att_rp, wcomb_rep.astype(jnp.float32),
                W_rc.T.astype(jnp.float32), W_rp.T.astype(jnp.float32))


# speedup vs baseline: 3.5692x; 1.0168x over previous
import jax
import jax.numpy as jnp
from jax import lax
from jax.experimental import pallas as pl
from jax.experimental.pallas import tpu as pltpu
from jax.experimental.pallas import tpu_sc as plsc

E = 320000
HID = 128
CHUNK = 256
GATH = 128
NCHUNKS = E // CHUNK
NC, NS = 2, 16
NW = NC * NS
NTAB = 35 * 9
NBUF = 3

BE = 8000


def _tc_body(rc_ref, rp_ref, wrc_ref, wrp_ref, out_ref):
    acc = jnp.dot(rc_ref[...], wrc_ref[...],
                  preferred_element_type=jnp.float32)
    acc = acc + jnp.dot(rp_ref[...], wrp_ref[...],
                        preferred_element_type=jnp.float32)
    out_ref[...] = acc


def _sc_body(et_hbm, rid_hbm, a_hbm, wcomb_hbm, out_hbm,
             et3, rid3, cidx3, rows3,
             isem0, isem1, isem2, gsem0, gsem1, gsem2,
             osem0, osem1, osem2):
    isem = (isem0, isem1, isem2)
    gsem = (gsem0, gsem1, gsem2)
    osem = (osem0, osem1, osem2)
    c = lax.axis_index("c")
    s = lax.axis_index("s")
    wid = s * NC + c

    n = (NCHUNKS - wid + NW - 1) // NW

    tab_base = wid * NTAB

    NG = CHUNK // GATH

    def in_slices(i, b):
        cid = wid + NW * i
        return ((et_hbm.at[pl.ds(cid * NG, NG)], et3.at[b]),
                (rid_hbm.at[pl.ds(cid * NG, NG)], rid3.at[b]),
                (a_hbm.at[pl.ds(cid * CHUNK, CHUNK)], rows3.at[b]))

    def issue_inputs(i, b):
        for src, dst in in_slices(i, b):
            pltpu.async_copy(src, dst, isem[b])

    def wait_inputs(b):
        for src, dst in in_slices(0, b):
            pltpu.make_async_copy(src, dst, isem[b]).wait()

    def compute_cidx(b):
        for h in range(CHUNK // GATH):
            for j in range(GATH // 16):
                sl = pl.ds(j * 16, 16)
                cidx3[b, h, sl] = et3[b, h, sl] * 9 + rid3[b, h, sl] + tab_base

    def issue_gather(b):
        for h in range(CHUNK // GATH):
            pltpu.async_copy(wcomb_hbm.at[cidx3.at[b, h]],
                             rows3.at[b, pl.ds(h * GATH, GATH)], gsem[b],
                             add=True)

    def wait_gather(b):
        pltpu.make_async_copy(out_hbm.at[pl.ds(0, CHUNK)], rows3.at[b],
                              gsem[b]).wait()

    def issue_out(i, b):
        cid = wid + NW * i
        pltpu.async_copy(rows3.at[b], out_hbm.at[pl.ds(cid * CHUNK, CHUNK)],
                         osem[b])

    def wait_out(b):
        pltpu.make_async_copy(rows3.at[b], out_hbm.at[pl.ds(0, CHUNK)],
                              osem[b]).wait()

    def process(i, b):
        bn = (b + 1) % NBUF
        bp = (b + 2) % NBUF

        @pl.when(i >= 1)
        def _():
            wait_out(bp)

        @pl.when(jnp.logical_and(i >= 1, i + 2 < n))
        def _():
            issue_inputs(i + 2, bp)

        wait_gather(b)
        issue_out(i, b)

        @pl.when(i + 1 < n)
        def _():
            wait_inputs(bn)
            compute_cidx(bn)
            issue_gather(bn)

    for src, dst in in_slices(0, 0):
        pltpu.sync_copy(src, dst)
    compute_cidx(0)
    issue_gather(0)
    issue_inputs(1, 1)
    issue_inputs(2, 2)

    def trip_body(p, carry):
        process(3 * p, 0)
        process(3 * p + 1, 1)
        process(3 * p + 2, 2)
        return carry

    lax.fori_loop(0, n // NBUF, trip_body, 0)

    @pl.when(n % NBUF == 1)
    def _():
        process(n - 1, 0)

    @pl.when(n % NBUF == 0)
    def _():
        wait_out(2)

    @pl.when(n % NBUF == 1)
    def _():
        wait_out(0)


@jax.jit
def _run(et, rid, rc, rp, wcomb, wrc_t, wrp_t):
    a = pl.pallas_call(
        _tc_body,
        grid=(E // BE,),
        in_specs=[
            pl.BlockSpec((BE, 2), lambda i: (i, 0)),
            pl.BlockSpec((BE, 3), lambda i: (i, 0)),
            pl.BlockSpec((2, HID), lambda i: (0, 0)),
            pl.BlockSpec((3, HID), lambda i: (0, 0)),
        ],
        out_specs=pl.BlockSpec((BE, HID), lambda i: (i, 0)),
        out_shape=jax.ShapeDtypeStruct((E, HID), jnp.float32),
    )(rc, rp, wrc_t, wrp_t)

    mesh = plsc.VectorSubcoreMesh(core_axis_name="c", subcore_axis_name="s",
                                  num_cores=NC, num_subcores=NS)
    fn = pl.kernel(
        _sc_body,
        out_type=jax.ShapeDtypeStruct((E, HID), jnp.float32),
        mesh=mesh,
        scratch_types=[
            pltpu.VMEM((NBUF, CHUNK // GATH, GATH), jnp.int32),
            pltpu.VMEM((NBUF, CHUNK // GATH, GATH), jnp.int32),
            pltpu.VMEM((NBUF, CHUNK // GATH, GATH), jnp.int32),
            pltpu.VMEM((NBUF, CHUNK, HID), jnp.float32),
            pltpu.SemaphoreType.DMA,
            pltpu.SemaphoreType.DMA,
            pltpu.SemaphoreType.DMA,
            pltpu.SemaphoreType.DMA,
            pltpu.SemaphoreType.DMA,
            pltpu.SemaphoreType.DMA,
            pltpu.SemaphoreType.DMA,
            pltpu.SemaphoreType.DMA,
            pltpu.SemaphoreType.DMA,
        ],
    )
    return fn(et, rid, a, wcomb)


def kernel(etype, rid, att_rc, att_rp, W_type, W_rid, W_rc, b_rc, W_rp, b_rp):
    wcomb = (W_type[:, None, :] + W_rid[None, :, :]).reshape(-1, HID)
    wcomb = wcomb + b_rc + b_rp
    wcomb_rep = jnp.tile(wcomb, (NW, 1))
    et128 = etype.astype(jnp.int32).reshape(E // GATH, GATH)
    rid128 = rid.astype(jnp.int32).reshape(E // GATH, GATH)
    return _run(et128, rid128,
                att_rc, att_rp, wcomb_rep.astype(jnp.float32),
                W_rc.T.astype(jnp.float32), W_rp.T.astype(jnp.float32))


# speedup vs baseline: 3.5701x; 1.0002x over previous
import jax
import jax.numpy as jnp
from jax import lax
from jax.experimental import pallas as pl
from jax.experimental.pallas import tpu as pltpu
from jax.experimental.pallas import tpu_sc as plsc

E = 320000
HID = 128
CHUNK = 256
GATH = 128
NCHUNKS = E // CHUNK
NC, NS = 2, 16
NW = NC * NS
NTAB = 35 * 9
NBUF = 3

BE = 16000


def _tc_body(rc_ref, rp_ref, wrc_ref, wrp_ref, out_ref):
    acc = jnp.dot(rc_ref[...], wrc_ref[...],
                  preferred_element_type=jnp.float32)
    acc = acc + jnp.dot(rp_ref[...], wrp_ref[...],
                        preferred_element_type=jnp.float32)
    out_ref[...] = acc


def _sc_body(et_hbm, rid_hbm, a_hbm, wcomb_hbm, out_hbm,
             et3, rid3, cidx3, rows3,
             isem0, isem1, isem2, gsem0, gsem1, gsem2,
             osem0, osem1, osem2):
    isem = (isem0, isem1, isem2)
    gsem = (gsem0, gsem1, gsem2)
    osem = (osem0, osem1, osem2)
    c = lax.axis_index("c")
    s = lax.axis_index("s")
    wid = s * NC + c

    n = (NCHUNKS - wid + NW - 1) // NW

    tab_base = wid * NTAB

    NG = CHUNK // GATH

    def in_slices(i, b):
        cid = wid + NW * i
        return ((et_hbm.at[pl.ds(cid * NG, NG)], et3.at[b]),
                (rid_hbm.at[pl.ds(cid * NG, NG)], rid3.at[b]),
                (a_hbm.at[pl.ds(cid * CHUNK, CHUNK)], rows3.at[b]))

    def issue_inputs(i, b):
        for src, dst in in_slices(i, b):
            pltpu.async_copy(src, dst, isem[b])

    def wait_inputs(b):
        for src, dst in in_slices(0, b):
            pltpu.make_async_copy(src, dst, isem[b]).wait()

    def compute_cidx(b):
        for h in range(CHUNK // GATH):
            for j in range(GATH // 16):
                sl = pl.ds(j * 16, 16)
                cidx3[b, h, sl] = et3[b, h, sl] * 9 + rid3[b, h, sl] + tab_base

    def issue_gather(b):
        for h in range(CHUNK // GATH):
            pltpu.async_copy(wcomb_hbm.at[cidx3.at[b, h]],
                             rows3.at[b, pl.ds(h * GATH, GATH)], gsem[b],
                             add=True)

    def wait_gather(b):
        pltpu.make_async_copy(out_hbm.at[pl.ds(0, CHUNK)], rows3.at[b],
                              gsem[b]).wait()

    def issue_out(i, b):
        cid = wid + NW * i
        pltpu.async_copy(rows3.at[b], out_hbm.at[pl.ds(cid * CHUNK, CHUNK)],
                         osem[b])

    def wait_out(b):
        pltpu.make_async_copy(rows3.at[b], out_hbm.at[pl.ds(0, CHUNK)],
                              osem[b]).wait()

    def process(i, b):
        bn = (b + 1) % NBUF
        bp = (b + 2) % NBUF

        @pl.when(i >= 1)
        def _():
            wait_out(bp)

        @pl.when(jnp.logical_and(i >= 1, i + 2 < n))
        def _():
            issue_inputs(i + 2, bp)

        wait_gather(b)
        issue_out(i, b)

        @pl.when(i + 1 < n)
        def _():
            wait_inputs(bn)
            compute_cidx(bn)
            issue_gather(bn)

    for src, dst in in_slices(0, 0):
        pltpu.sync_copy(src, dst)
    compute_cidx(0)
    issue_gather(0)
    issue_inputs(1, 1)
    issue_inputs(2, 2)

    def trip_body(p, carry):
        process(3 * p, 0)
        process(3 * p + 1, 1)
        process(3 * p + 2, 2)
        return carry

    lax.fori_loop(0, n // NBUF, trip_body, 0)

    @pl.when(n % NBUF == 1)
    def _():
        process(n - 1, 0)

    @pl.when(n % NBUF == 0)
    def _():
        wait_out(2)

    @pl.when(n % NBUF == 1)
    def _():
        wait_out(0)


@jax.jit
def _run(et, rid, rc, rp, wcomb, wrc_t, wrp_t):
    a = pl.pallas_call(
        _tc_body,
        grid=(E // BE,),
        in_specs=[
            pl.BlockSpec((BE, 2), lambda i: (i, 0)),
            pl.BlockSpec((BE, 3), lambda i: (i, 0)),
            pl.BlockSpec((2, HID), lambda i: (0, 0)),
            pl.BlockSpec((3, HID), lambda i: (0, 0)),
        ],
        out_specs=pl.BlockSpec((BE, HID), lambda i: (i, 0)),
        out_shape=jax.ShapeDtypeStruct((E, HID), jnp.float32),
    )(rc, rp, wrc_t, wrp_t)

    mesh = plsc.VectorSubcoreMesh(core_axis_name="c", subcore_axis_name="s",
                                  num_cores=NC, num_subcores=NS)
    fn = pl.kernel(
        _sc_body,
        out_type=jax.ShapeDtypeStruct((E, HID), jnp.float32),
        mesh=mesh,
        scratch_types=[
            pltpu.VMEM((NBUF, CHUNK // GATH, GATH), jnp.int32),
            pltpu.VMEM((NBUF, CHUNK // GATH, GATH), jnp.int32),
            pltpu.VMEM((NBUF, CHUNK // GATH, GATH), jnp.int32),
            pltpu.VMEM((NBUF, CHUNK, HID), jnp.float32),
            pltpu.SemaphoreType.DMA,
            pltpu.SemaphoreType.DMA,
            pltpu.SemaphoreType.DMA,
            pltpu.SemaphoreType.DMA,
            pltpu.SemaphoreType.DMA,
            pltpu.SemaphoreType.DMA,
            pltpu.SemaphoreType.DMA,
            pltpu.SemaphoreType.DMA,
            pltpu.SemaphoreType.DMA,
        ],
    )
    return fn(et, rid, a, wcomb)


def kernel(etype, rid, att_rc, att_rp, W_type, W_rid, W_rc, b_rc, W_rp, b_rp):
    wcomb = (W_type[:, None, :] + W_rid[None, :, :]).reshape(-1, HID)
    wcomb = wcomb + b_rc + b_rp
    wcomb_rep = jnp.tile(wcomb, (NW, 1))
    et128 = etype.astype(jnp.int32).reshape(E // GATH, GATH)
    rid128 = rid.astype(jnp.int32).reshape(E // GATH, GATH)
    return _run(et128, rid128,
                att_rc, att_rp, wcomb_rep.astype(jnp.float32),
                W_rc.T.astype(jnp.float32), W_rp.T.astype(jnp.float32))
